# baseline (speedup 1.0000x reference)
import functools

import jax
import jax.numpy as jnp
from jax.experimental import pallas as pl
from jax.experimental.pallas import tpu as pltpu

_INTERPRET = False
_K = 2


def _moe_kernel_body(TB, K, DFF, D, CH, gk_ref, tok_ref, w1s_ref, b1s_ref,
                     w2s_ref, b2s_ref, out_ref):
    pid = pl.program_id(0)
    L = tok_ref.shape[2]
    for c in range(L // CH):
        sl = pl.ds(c * CH, CH)
        for i in range(TB):
            t = pid * TB + i
            tok_c = tok_ref[i, :, sl]
            h = jnp.dot(w1s_ref[i], tok_c,
                        preferred_element_type=jnp.float32)
            h = jnp.maximum(h + b1s_ref[i], 0.0)
            y0 = jnp.dot(w2s_ref[i, :D], h[:DFF],
                         preferred_element_type=jnp.float32)
            y0 = y0 + b2s_ref[i, :D]
            y1 = jnp.dot(w2s_ref[i, D:], h[DFF:],
                         preferred_element_type=jnp.float32)
            y1 = y1 + b2s_ref[i, D:]
            out_ref[i, :, sl] = gk_ref[0, t] * y0 + gk_ref[1, t] * y1


def _moe_pallas(tokT, gk, w1s, b1s, w2s, b2s):
    T, D, L = tokT.shape
    KDFF = w1s.shape[1]
    K = gk.shape[1]
    DFF = KDFF // K
    gkT = jnp.swapaxes(gk, 0, 1)
    TB = 16
    CH = 512
    body = functools.partial(_moe_kernel_body, TB, K, DFF, D, CH)
    return pl.pallas_call(
        body,
        grid=(T // TB,),
        in_specs=[
            pl.BlockSpec(memory_space=pltpu.MemorySpace.SMEM),
            pl.BlockSpec((TB, D, L), lambda i: (i, 0, 0)),
            pl.BlockSpec((TB, KDFF, D), lambda i: (i, 0, 0)),
            pl.BlockSpec((TB, KDFF, 1), lambda i: (i, 0, 0)),
            pl.BlockSpec((TB, K * D, DFF), lambda i: (i, 0, 0)),
            pl.BlockSpec((TB, K * D, 1), lambda i: (i, 0, 0)),
        ],
        out_specs=pl.BlockSpec((TB, D, L), lambda i: (i, 0, 0)),
        out_shape=jax.ShapeDtypeStruct((T, D, L), jnp.float32),
        compiler_params=pltpu.CompilerParams(
            dimension_semantics=("parallel",)),
        interpret=_INTERPRET,
    )(gkT, tokT, w1s, b1s, w2s, b2s)


def _proj_kernel_body(x_ref, w_ref, b_ref, out_ref):
    out_ref[...] = jnp.dot(x_ref[...], w_ref[...],
                           preferred_element_type=jnp.float32) + b_ref[...]


def _proj_pallas(flat, w, b):
    T, LD = flat.shape
    P = w.shape[1]
    TBP = 128
    return pl.pallas_call(
        _proj_kernel_body,
        grid=(T // TBP,),
        in_specs=[
            pl.BlockSpec((TBP, LD), lambda i: (i, 0)),
            pl.BlockSpec((LD, P), lambda i: (0, 0)),
            pl.BlockSpec((1, P), lambda i: (0, 0)),
        ],
        out_specs=pl.BlockSpec((TBP, P), lambda i: (i, 0)),
        out_shape=jax.ShapeDtypeStruct((T, P), jnp.float32),
        compiler_params=pltpu.CompilerParams(
            dimension_semantics=("parallel",)),
        interpret=_INTERPRET,
    )(flat, w, b)


def _cv_sq(v):
    eps = 1e-10
    return v.var() / (v.mean() ** 2 + eps)


def kernel(x, params):
    B, L, N = x.shape
    D = params['start_W'].shape[1]
    T = B * N
    P = params['proj_W'].shape[1]
    K = _K

    mean = x.mean(axis=1, keepdims=True)
    std = jnp.sqrt(x.var(axis=1, keepdims=True) + 1e-5)
    xn = (x - mean) / std
    out = xn[..., None] * params['start_W'][0] + params['start_b']
    balance = jnp.float32(0.0)

    for lp in params['layers']:
        tok = out.transpose(0, 2, 1, 3).reshape(T, L, D)
        feat = tok.mean(axis=1)
        logits = feat @ lp['w_gate']
        topv, topi = jax.lax.top_k(logits, K)
        gk = jax.nn.softmax(topv, axis=-1)
        gates = jnp.zeros((T, logits.shape[1]), dtype=x.dtype).at[
            jnp.arange(T)[:, None], topi].set(gk)
        importance = gates.sum(axis=0)
        load = (gates > 0).astype(jnp.float32).sum(axis=0)
        balance = balance + _cv_sq(importance) + _cv_sq(load)

        tokT = out.transpose(0, 2, 3, 1).reshape(T, D, L)
        w1T = jnp.swapaxes(lp['W1'], 1, 2)
        w2T = jnp.swapaxes(lp['W2'], 1, 2)
        DFF = w1T.shape[1]
        w1s = w1T[topi].reshape(T, K * DFF, D)
        b1s = lp['b1'][topi].reshape(T, K * DFF, 1)
        w2s = w2T[topi].reshape(T, K * D, DFF)
        b2s = lp['b2'][topi].reshape(T, K * D, 1)
        moeT = _moe_pallas(tokT, gk, w1s, b1s, w2s, b2s)
        moe = moeT.reshape(B, N, D, L).transpose(0, 3, 1, 2)
        out = out + moe

    flat = out.transpose(0, 2, 1, 3).reshape(T, L * D)
    proj = _proj_pallas(flat, params['proj_W'],
                        params['proj_b'][None, :]).reshape(B, N, P)
    yhat = proj.transpose(0, 2, 1)
    yhat = yhat * std + mean
    con = jnp.mean(jnp.stack([jnp.float32(0.0)] * len(params['layers'])))
    return yhat, balance, con


# speedup vs baseline: 2.1687x; 2.1687x over previous
import functools

import jax
import jax.numpy as jnp
from jax.experimental import pallas as pl
from jax.experimental.pallas import tpu as pltpu

_INTERPRET = False
_K = 2


def _moe_kernel_body(TB, K, topi_ref, gk_ref, tok_ref, w1_ref, b1_ref,
                     w2_ref, b2_ref, out_ref, h_ref):
    pid = pl.program_id(0)
    DFF = w1_ref.shape[1]
    for i in range(TB):
        t = pid * TB + i
        e0 = topi_ref[0, t]
        e1 = topi_ref[1, t]
        w1s = jnp.concatenate([w1_ref[e0], w1_ref[e1]], axis=0)
        h_ref[i] = jnp.dot(w1s, tok_ref[i],
                           preferred_element_type=jnp.float32)
    for i in range(TB):
        t = pid * TB + i
        e0 = topi_ref[0, t]
        e1 = topi_ref[1, t]
        b1s = jnp.concatenate([b1_ref[e0], b1_ref[e1]], axis=0)
        h = jnp.maximum(h_ref[i] + b1s, 0.0)
        y0 = jnp.dot(w2_ref[e0], h[:DFF],
                     preferred_element_type=jnp.float32)
        y0 = y0 + b2_ref[e0]
        y1 = jnp.dot(w2_ref[e1], h[DFF:],
                     preferred_element_type=jnp.float32)
        y1 = y1 + b2_ref[e1]
        out_ref[i] = gk_ref[0, t] * y0 + gk_ref[1, t] * y1


def _moe_pallas(tokT, topi, gk, w1T, b1, w2T, b2):
    T, D, L = tokT.shape
    E, DFF, _ = w1T.shape
    K = topi.shape[1]
    topiT = jnp.swapaxes(topi, 0, 1)
    gkT = jnp.swapaxes(gk, 0, 1)
    TB = 16
    body = functools.partial(_moe_kernel_body, TB, K)
    return pl.pallas_call(
        body,
        grid=(T // TB,),
        in_specs=[
            pl.BlockSpec(memory_space=pltpu.MemorySpace.SMEM),
            pl.BlockSpec(memory_space=pltpu.MemorySpace.SMEM),
            pl.BlockSpec((TB, D, L), lambda i: (i, 0, 0)),
            pl.BlockSpec((E, DFF, D), lambda i: (0, 0, 0)),
            pl.BlockSpec((E, DFF, 1), lambda i: (0, 0, 0)),
            pl.BlockSpec((E, D, DFF), lambda i: (0, 0, 0)),
            pl.BlockSpec((E, D, 1), lambda i: (0, 0, 0)),
        ],
        out_specs=pl.BlockSpec((TB, D, L), lambda i: (i, 0, 0)),
        out_shape=jax.ShapeDtypeStruct((T, D, L), jnp.float32),
        scratch_shapes=[pltpu.VMEM((TB, K * DFF, L), jnp.float32)],
        interpret=_INTERPRET,
    )(topiT, gkT, tokT, w1T, b1, w2T, b2)


def _proj_kernel_body(x_ref, w_ref, b_ref, out_ref):
    out_ref[...] = jnp.dot(x_ref[...], w_ref[...],
                           preferred_element_type=jnp.float32) + b_ref[...]


def _proj_pallas(flat, w, b):
    T, LD = flat.shape
    P = w.shape[1]
    TBP = 128
    return pl.pallas_call(
        _proj_kernel_body,
        grid=(T // TBP,),
        in_specs=[
            pl.BlockSpec((TBP, LD), lambda i: (i, 0)),
            pl.BlockSpec((LD, P), lambda i: (0, 0)),
            pl.BlockSpec((1, P), lambda i: (0, 0)),
        ],
        out_specs=pl.BlockSpec((TBP, P), lambda i: (i, 0)),
        out_shape=jax.ShapeDtypeStruct((T, P), jnp.float32),
        compiler_params=pltpu.CompilerParams(
            dimension_semantics=("parallel",)),
        interpret=_INTERPRET,
    )(flat, w, b)


def _cv_sq(v):
    eps = 1e-10
    return v.var() / (v.mean() ** 2 + eps)


def kernel(x, params):
    B, L, N = x.shape
    D = params['start_W'].shape[1]
    T = B * N
    P = params['proj_W'].shape[1]
    K = _K

    mean = x.mean(axis=1, keepdims=True)
    std = jnp.sqrt(x.var(axis=1, keepdims=True) + 1e-5)
    xn = (x - mean) / std
    out = xn[..., None] * params['start_W'][0] + params['start_b']
    balance = jnp.float32(0.0)

    for lp in params['layers']:
        tok = out.transpose(0, 2, 1, 3).reshape(T, L, D)
        feat = tok.mean(axis=1)
        logits = feat @ lp['w_gate']
        topv, topi = jax.lax.top_k(logits, K)
        gk = jax.nn.softmax(topv, axis=-1)
        gates = jnp.zeros((T, logits.shape[1]), dtype=x.dtype).at[
            jnp.arange(T)[:, None], topi].set(gk)
        importance = gates.sum(axis=0)
        load = (gates > 0).astype(jnp.float32).sum(axis=0)
        balance = balance + _cv_sq(importance) + _cv_sq(load)

        tokT = out.transpose(0, 2, 3, 1).reshape(T, D, L)
        w1T = jnp.swapaxes(lp['W1'], 1, 2)
        w2T = jnp.swapaxes(lp['W2'], 1, 2)
        b1c = lp['b1'][:, :, None]
        b2c = lp['b2'][:, :, None]
        moeT = _moe_pallas(tokT, topi, gk, w1T, b1c, w2T, b2c)
        moe = moeT.reshape(B, N, D, L).transpose(0, 3, 1, 2)
        out = out + moe

    flat = out.transpose(0, 2, 1, 3).reshape(T, L * D)
    proj = _proj_pallas(flat, params['proj_W'],
                        params['proj_b'][None, :]).reshape(B, N, P)
    yhat = proj.transpose(0, 2, 1)
    yhat = yhat * std + mean
    con = jnp.mean(jnp.stack([jnp.float32(0.0)] * len(params['layers'])))
    return yhat, balance, con


# speedup vs baseline: 2.3766x; 1.0959x over previous
import functools

import jax
import jax.numpy as jnp
from jax.experimental import pallas as pl
from jax.experimental.pallas import tpu as pltpu

_INTERPRET = False
_K = 2


def _moe_kernel_body(TB, K, topi_ref, gk_ref, tok_ref, w1_ref, b1_ref,
                     w2_ref, b2_ref, out_ref, h_ref):
    pid = pl.program_id(0)
    DFF = w1_ref.shape[1]
    for i in range(TB):
        t = pid * TB + i
        e0 = topi_ref[0, t]
        e1 = topi_ref[1, t]
        w1s = jnp.concatenate([w1_ref[e0], w1_ref[e1]], axis=0)
        h_ref[i] = jnp.dot(w1s, tok_ref[i],
                           preferred_element_type=jnp.float32)
    for i in range(TB):
        t = pid * TB + i
        e0 = topi_ref[0, t]
        e1 = topi_ref[1, t]
        b1s = jnp.concatenate([b1_ref[e0], b1_ref[e1]], axis=0)
        h = jnp.maximum(h_ref[i] + b1s, 0.0)
        y0 = jnp.dot(w2_ref[e0], h[:DFF],
                     preferred_element_type=jnp.float32)
        y0 = y0 + b2_ref[e0]
        y1 = jnp.dot(w2_ref[e1], h[DFF:],
                     preferred_element_type=jnp.float32)
        y1 = y1 + b2_ref[e1]
        out_ref[i] = gk_ref[0, t] * y0 + gk_ref[1, t] * y1


def _moe_pallas(tokT, topi, gk, w1T, b1, w2T, b2):
    T, D, L = tokT.shape
    E, DFF, _ = w1T.shape
    K = topi.shape[1]
    topiT = jnp.swapaxes(topi, 0, 1)
    gkT = jnp.swapaxes(gk, 0, 1)
    TB = 32
    body = functools.partial(_moe_kernel_body, TB, K)
    return pl.pallas_call(
        body,
        grid=(T // TB,),
        in_specs=[
            pl.BlockSpec(memory_space=pltpu.MemorySpace.SMEM),
            pl.BlockSpec(memory_space=pltpu.MemorySpace.SMEM),
            pl.BlockSpec((TB, D, L), lambda i: (i, 0, 0)),
            pl.BlockSpec((E, DFF, D), lambda i: (0, 0, 0)),
            pl.BlockSpec((E, DFF, 1), lambda i: (0, 0, 0)),
            pl.BlockSpec((E, D, DFF), lambda i: (0, 0, 0)),
            pl.BlockSpec((E, D, 1), lambda i: (0, 0, 0)),
        ],
        out_specs=pl.BlockSpec((TB, D, L), lambda i: (i, 0, 0)),
        out_shape=jax.ShapeDtypeStruct((T, D, L), jnp.float32),
        scratch_shapes=[pltpu.VMEM((TB, K * DFF, L), jnp.float32)],
        interpret=_INTERPRET,
    )(topiT, gkT, tokT, w1T, b1, w2T, b2)


def _proj_kernel_body(x_ref, w_ref, b_ref, out_ref):
    out_ref[...] = jnp.dot(x_ref[...], w_ref[...],
                           preferred_element_type=jnp.float32) + b_ref[...]


def _proj_pallas(flat, w, b):
    T, LD = flat.shape
    P = w.shape[1]
    TBP = 128
    return pl.pallas_call(
        _proj_kernel_body,
        grid=(T // TBP,),
        in_specs=[
            pl.BlockSpec((TBP, LD), lambda i: (i, 0)),
            pl.BlockSpec((LD, P), lambda i: (0, 0)),
            pl.BlockSpec((1, P), lambda i: (0, 0)),
        ],
        out_specs=pl.BlockSpec((TBP, P), lambda i: (i, 0)),
        out_shape=jax.ShapeDtypeStruct((T, P), jnp.float32),
        compiler_params=pltpu.CompilerParams(
            dimension_semantics=("parallel",)),
        interpret=_INTERPRET,
    )(flat, w, b)


def _cv_sq(v):
    eps = 1e-10
    return v.var() / (v.mean() ** 2 + eps)


def kernel(x, params):
    B, L, N = x.shape
    D = params['start_W'].shape[1]
    T = B * N
    P = params['proj_W'].shape[1]
    K = _K

    mean = x.mean(axis=1, keepdims=True)
    std = jnp.sqrt(x.var(axis=1, keepdims=True) + 1e-5)
    xn = (x - mean) / std
    out = xn[..., None] * params['start_W'][0] + params['start_b']
    balance = jnp.float32(0.0)

    for lp in params['layers']:
        tok = out.transpose(0, 2, 1, 3).reshape(T, L, D)
        feat = tok.mean(axis=1)
        logits = feat @ lp['w_gate']
        topv, topi = jax.lax.top_k(logits, K)
        gk = jax.nn.softmax(topv, axis=-1)
        gates = jnp.zeros((T, logits.shape[1]), dtype=x.dtype).at[
            jnp.arange(T)[:, None], topi].set(gk)
        importance = gates.sum(axis=0)
        load = (gates > 0).astype(jnp.float32).sum(axis=0)
        balance = balance + _cv_sq(importance) + _cv_sq(load)

        tokT = out.transpose(0, 2, 3, 1).reshape(T, D, L)
        w1T = jnp.swapaxes(lp['W1'], 1, 2)
        w2T = jnp.swapaxes(lp['W2'], 1, 2)
        b1c = lp['b1'][:, :, None]
        b2c = lp['b2'][:, :, None]
        moeT = _moe_pallas(tokT, topi, gk, w1T, b1c, w2T, b2c)
        moe = moeT.reshape(B, N, D, L).transpose(0, 3, 1, 2)
        out = out + moe

    flat = out.transpose(0, 2, 1, 3).reshape(T, L * D)
    proj = _proj_pallas(flat, params['proj_W'],
                        params['proj_b'][None, :]).reshape(B, N, P)
    yhat = proj.transpose(0, 2, 1)
    yhat = yhat * std + mean
    con = jnp.mean(jnp.stack([jnp.float32(0.0)] * len(params['layers'])))
    return yhat, balance, con


# speedup vs baseline: 2.4934x; 1.0492x over previous
import functools

import jax
import jax.numpy as jnp
from jax.experimental import pallas as pl
from jax.experimental.pallas import tpu as pltpu

_INTERPRET = False
_K = 2


def _moe_kernel_body(TB, K, topi_ref, gk_ref, tok_ref, w1_ref, b1_ref,
                     w2_ref, b2_ref, out_ref, h_ref):
    pid = pl.program_id(0)
    DFF = w1_ref.shape[1]
    for i in range(TB):
        t = pid * TB + i
        e0 = topi_ref[0, t]
        e1 = topi_ref[1, t]
        w1s = jnp.concatenate([w1_ref[e0], w1_ref[e1]], axis=0)
        h_ref[i] = jnp.dot(w1s, tok_ref[i],
                           preferred_element_type=jnp.float32)
    for i in range(TB):
        t = pid * TB + i
        e0 = topi_ref[0, t]
        e1 = topi_ref[1, t]
        b1s = jnp.concatenate([b1_ref[e0], b1_ref[e1]], axis=0)
        h = jnp.maximum(h_ref[i] + b1s, 0.0)
        y0 = jnp.dot(w2_ref[e0], h[:DFF],
                     preferred_element_type=jnp.float32)
        y0 = y0 + b2_ref[e0]
        y1 = jnp.dot(w2_ref[e1], h[DFF:],
                     preferred_element_type=jnp.float32)
        y1 = y1 + b2_ref[e1]
        out_ref[i] = gk_ref[0, t] * y0 + gk_ref[1, t] * y1


def _moe_pallas(tokT, topi, gk, w1T, b1, w2T, b2):
    T, D, L = tokT.shape
    E, DFF, _ = w1T.shape
    K = topi.shape[1]
    topiT = jnp.swapaxes(topi, 0, 1)
    gkT = jnp.swapaxes(gk, 0, 1)
    TB = 64
    body = functools.partial(_moe_kernel_body, TB, K)
    return pl.pallas_call(
        body,
        grid=(T // TB,),
        in_specs=[
            pl.BlockSpec(memory_space=pltpu.MemorySpace.SMEM),
            pl.BlockSpec(memory_space=pltpu.MemorySpace.SMEM),
            pl.BlockSpec((TB, D, L), lambda i: (i, 0, 0)),
            pl.BlockSpec((E, DFF, D), lambda i: (0, 0, 0)),
            pl.BlockSpec((E, DFF, 1), lambda i: (0, 0, 0)),
            pl.BlockSpec((E, D, DFF), lambda i: (0, 0, 0)),
            pl.BlockSpec((E, D, 1), lambda i: (0, 0, 0)),
        ],
        out_specs=pl.BlockSpec((TB, D, L), lambda i: (i, 0, 0)),
        out_shape=jax.ShapeDtypeStruct((T, D, L), jnp.float32),
        scratch_shapes=[pltpu.VMEM((TB, K * DFF, L), jnp.float32)],
        interpret=_INTERPRET,
    )(topiT, gkT, tokT, w1T, b1, w2T, b2)


def _proj_kernel_body(x_ref, w_ref, b_ref, out_ref):
    out_ref[...] = jnp.dot(x_ref[...], w_ref[...],
                           preferred_element_type=jnp.float32) + b_ref[...]


def _proj_pallas(flat, w, b):
    T, LD = flat.shape
    P = w.shape[1]
    TBP = 128
    return pl.pallas_call(
        _proj_kernel_body,
        grid=(T // TBP,),
        in_specs=[
            pl.BlockSpec((TBP, LD), lambda i: (i, 0)),
            pl.BlockSpec((LD, P), lambda i: (0, 0)),
            pl.BlockSpec((1, P), lambda i: (0, 0)),
        ],
        out_specs=pl.BlockSpec((TBP, P), lambda i: (i, 0)),
        out_shape=jax.ShapeDtypeStruct((T, P), jnp.float32),
        compiler_params=pltpu.CompilerParams(
            dimension_semantics=("parallel",)),
        interpret=_INTERPRET,
    )(flat, w, b)


def _cv_sq(v):
    eps = 1e-10
    return v.var() / (v.mean() ** 2 + eps)


def kernel(x, params):
    B, L, N = x.shape
    D = params['start_W'].shape[1]
    T = B * N
    P = params['proj_W'].shape[1]
    K = _K

    mean = x.mean(axis=1, keepdims=True)
    std = jnp.sqrt(x.var(axis=1, keepdims=True) + 1e-5)
    xn = (x - mean) / std
    out = xn[..., None] * params['start_W'][0] + params['start_b']
    balance = jnp.float32(0.0)

    for lp in params['layers']:
        tok = out.transpose(0, 2, 1, 3).reshape(T, L, D)
        feat = tok.mean(axis=1)
        logits = feat @ lp['w_gate']
        topv, topi = jax.lax.top_k(logits, K)
        gk = jax.nn.softmax(topv, axis=-1)
        gates = jnp.zeros((T, logits.shape[1]), dtype=x.dtype).at[
            jnp.arange(T)[:, None], topi].set(gk)
        importance = gates.sum(axis=0)
        load = (gates > 0).astype(jnp.float32).sum(axis=0)
        balance = balance + _cv_sq(importance) + _cv_sq(load)

        tokT = out.transpose(0, 2, 3, 1).reshape(T, D, L)
        w1T = jnp.swapaxes(lp['W1'], 1, 2)
        w2T = jnp.swapaxes(lp['W2'], 1, 2)
        b1c = lp['b1'][:, :, None]
        b2c = lp['b2'][:, :, None]
        moeT = _moe_pallas(tokT, topi, gk, w1T, b1c, w2T, b2c)
        moe = moeT.reshape(B, N, D, L).transpose(0, 3, 1, 2)
        out = out + moe

    flat = out.transpose(0, 2, 1, 3).reshape(T, L * D)
    proj = _proj_pallas(flat, params['proj_W'],
                        params['proj_b'][None, :]).reshape(B, N, P)
    yhat = proj.transpose(0, 2, 1)
    yhat = yhat * std + mean
    con = jnp.mean(jnp.stack([jnp.float32(0.0)] * len(params['layers'])))
    return yhat, balance, con
